# Initial kernel scaffold; baseline (speedup 1.0000x reference)
import jax
import jax.numpy as jnp
from jax.experimental import pallas as pl


def kernel(vectors, node_feats, radial_embedding, receivers, W_proj, W_r, b_r, ln_g, ln_b):
    raise NotImplementedError("write your pallas kernel here")


# speedup vs baseline: 1.4331x; 1.4331x over previous
import functools

import jax
import jax.numpy as jnp
from jax import lax
from jax.experimental import pallas as pl
from jax.experimental.pallas import tpu as pltpu
from jax.experimental.pallas import tpu_sc as plsc

N_NODES = 10000
DEG = 16
D_FEAT = 128
N_RADIAL = 8
TP_CH = 32
SH_DIM = 8
MSG_DIM = D_FEAT + TP_CH * SH_DIM
E_TOTAL = N_NODES * DEG

B_NODES = 200
B_EDGES = B_NODES * DEG
N_BLOCKS = N_NODES // B_NODES

_SQ3 = 3.0 ** 0.5
_SQ15 = 15.0 ** 0.5
_SQ5H = (5.0 ** 0.5) / 2.0
_SQ15H = _SQ15 / 2.0
_SCALE = 0.25


def _tc_body(v_ref, r_ref, feats_ref, w2_ref, t_ref, wr_ref, br_ref, g_ref,
             b_ref, out_ref):
    feats = feats_ref[...]
    p2 = jnp.dot(feats, w2_ref[...], preferred_element_type=jnp.float32)
    p2e = jnp.reshape(jnp.broadcast_to(p2[:, None, :], (B_NODES, DEG, 2 * D_FEAT)),
                      (B_EDGES, 2 * D_FEAT))
    fe = jnp.reshape(jnp.broadcast_to(feats[:, None, :], (B_NODES, DEG, D_FEAT)),
                     (B_EDGES, D_FEAT))
    v = v_ref[...]
    x = v[:, 0:1]
    y = v[:, 1:2]
    z = v[:, 2:3]
    inv = 1.0 / (jnp.sqrt(x * x + y * y + z * z) + 1e-9)
    x = x * inv
    y = y * inv
    z = z * inv
    sh8 = jnp.concatenate([
        _SQ3 * x, _SQ3 * y, _SQ3 * z,
        _SQ15 * x * y, _SQ15 * y * z, _SQ5H * (3.0 * z * z - 1.0),
        _SQ15 * x * z, _SQ15H * (x * x - y * y),
    ], axis=1)
    shm = jnp.dot(sh8, t_ref[...], preferred_element_type=jnp.float32)
    radial = jnp.dot(r_ref[...], wr_ref[...],
                     preferred_element_type=jnp.float32) + br_ref[...]
    mu = jnp.mean(radial, axis=1, keepdims=True)
    d = radial - mu
    var = jnp.mean(d * d, axis=1, keepdims=True)
    radial = (d * lax.rsqrt(var + 1e-6) * g_ref[...] + b_ref[...]) * _SCALE
    out_ref[...] = jnp.concatenate(
        [fe * radial[:, :D_FEAT], p2e * shm * radial[:, D_FEAT:]], axis=1)


def _messages_tc(vecs2, rad2, node_feats, w2, t, wr, br2, g2, b2):
    return pl.pallas_call(
        _tc_body,
        grid=(N_BLOCKS,),
        in_specs=[
            pl.BlockSpec((B_EDGES, 3), lambda i: (i, 0)),
            pl.BlockSpec((B_EDGES, N_RADIAL), lambda i: (i, 0)),
            pl.BlockSpec((B_NODES, D_FEAT), lambda i: (i, 0)),
            pl.BlockSpec((D_FEAT, 2 * D_FEAT), lambda i: (0, 0)),
            pl.BlockSpec((SH_DIM, 2 * D_FEAT), lambda i: (0, 0)),
            pl.BlockSpec((N_RADIAL, MSG_DIM), lambda i: (0, 0)),
            pl.BlockSpec((1, MSG_DIM), lambda i: (0, 0)),
            pl.BlockSpec((1, MSG_DIM), lambda i: (0, 0)),
            pl.BlockSpec((1, MSG_DIM), lambda i: (0, 0)),
        ],
        out_specs=pl.BlockSpec((B_EDGES, MSG_DIM), lambda i: (i, 0)),
        out_shape=jax.ShapeDtypeStruct((E_TOTAL, MSG_DIM), jnp.float32),
    )(vecs2, rad2, node_feats, w2, t, wr, br2, g2, b2)


HALF = N_NODES // 2
ACC_ROWS = 5008
EDGES_PER_TILE = E_TOTAL // 16
CHUNK = 16
IDS_BLOCK = 2000
WB_ROWS = 312

@functools.cache
def _make_scatter_sc():
    mesh = plsc.VectorSubcoreMesh(core_axis_name="c", subcore_axis_name="s",
                                  num_cores=2, num_subcores=16)
    return pl.kernel(
        _sc_body,
        out_type=jax.ShapeDtypeStruct((N_NODES, MSG_DIM), jnp.float32),
        mesh=mesh,
        scratch_types=[
            pltpu.VMEM_SHARED((ACC_ROWS, MSG_DIM), jnp.float32),
            pltpu.VMEM((CHUNK, MSG_DIM), jnp.float32),
            pltpu.VMEM((IDS_BLOCK,), jnp.int32),
            pltpu.VMEM((CHUNK,), jnp.int32),
        ],
    )


def _sc_body(msgs_hbm, recv_hbm, out_hbm, acc, mbuf, idbuf, idxl):
    cid = lax.axis_index("c")
    sid = lax.axis_index("s")
    lo = cid * HALF
    zero16 = jnp.zeros((16,), jnp.float32)
    for r in range(CHUNK):
        for cch in range(MSG_DIM // 16):
            mbuf[r, pl.ds(cch * 16, 16)] = zero16
    for k in range(20):
        start = jnp.minimum((sid * 20 + k) * 16, ACC_ROWS - 16)
        start = pl.multiple_of(start, 16)
        pltpu.sync_copy(mbuf, acc.at[pl.ds(start, 16)])
    plsc.subcore_barrier()

    base0 = sid * EDGES_PER_TILE
    dump = HALF + (sid % 8)

    def body(k, carry):
        o = k // (IDS_BLOCK // CHUNK)
        j = k % (IDS_BLOCK // CHUNK)

        @pl.when(j == 0)
        def _():
            pltpu.sync_copy(recv_hbm.at[pl.ds(base0 + o * IDS_BLOCK,
                                              IDS_BLOCK)], idbuf)

        base = base0 + k * CHUNK
        pltpu.sync_copy(msgs_hbm.at[pl.ds(base, CHUNK)], mbuf)
        loc = idbuf[pl.ds(j * CHUNK, 16)] - lo
        oob = (loc < 0) | (loc >= HALF)
        idxl[...] = jnp.where(oob, dump, loc)
        pltpu.sync_copy(mbuf, acc.at[idxl], add=True)
        return carry

    lax.fori_loop(0, EDGES_PER_TILE // CHUNK, body, 0)
    plsc.subcore_barrier()

    base_r = sid * WB_ROWS

    @pl.when(sid < 15)
    def _():
        pltpu.sync_copy(acc.at[pl.ds(base_r, WB_ROWS)],
                        out_hbm.at[pl.ds(lo + base_r, WB_ROWS)])

    @pl.when(sid == 15)
    def _():
        pltpu.sync_copy(acc.at[pl.ds(15 * WB_ROWS, HALF - 15 * WB_ROWS)],
                        out_hbm.at[pl.ds(lo + 15 * WB_ROWS, HALF - 15 * WB_ROWS)])


def kernel(vectors, node_feats, radial_embedding, receivers, W_proj, W_r,
           b_r, ln_g, ln_b):
    vecs2 = vectors.reshape(E_TOTAL, 3)
    rad2 = radial_embedding.reshape(E_TOTAL, N_RADIAL)
    recv = receivers.reshape(E_TOTAL).astype(jnp.int32)
    w2 = jnp.repeat(W_proj, SH_DIM, axis=1)
    t = jnp.tile(jnp.eye(SH_DIM, dtype=jnp.float32), (1, TP_CH))
    msgs = _messages_tc(vecs2, rad2, node_feats, w2, t, W_r,
                        b_r.reshape(1, MSG_DIM), ln_g.reshape(1, MSG_DIM),
                        ln_b.reshape(1, MSG_DIM))
    return jnp.zeros((N_NODES, MSG_DIM), jnp.float32).at[recv].add(msgs)
